# baseline (speedup 1.0000x reference)
import jax
import jax.numpy as jnp
from jax.experimental import pallas as pl
from jax.experimental.pallas import tpu as pltpu

_N = 4


def _copy_kernel(x_ref, o_ref, *rest):
    bufs = rest[:_N]
    sin = rest[_N:2 * _N]
    sout = rest[2 * _N:]
    h = x_ref.shape[0] // _N
    ins = [
        pltpu.make_async_copy(x_ref.at[pl.ds(i * h, h)], bufs[i], sin[i])
        for i in range(_N)
    ]
    outs = [
        pltpu.make_async_copy(bufs[i], o_ref.at[pl.ds(i * h, h)], sout[i])
        for i in range(_N)
    ]
    for c in ins:
        c.start()
    for i in range(_N):
        ins[i].wait()
        outs[i].start()
    for c in outs:
        c.wait()


def kernel(input):
    t, b, s = input.shape
    return pl.pallas_call(
        _copy_kernel,
        out_shape=jax.ShapeDtypeStruct(input.shape, input.dtype),
        in_specs=[pl.BlockSpec(memory_space=pl.ANY)],
        out_specs=pl.BlockSpec(memory_space=pl.ANY),
        scratch_shapes=(
            [pltpu.VMEM((t // _N, b, s), input.dtype) for _ in range(_N)]
            + [pltpu.SemaphoreType.DMA] * (2 * _N)
        ),
    )(input)
